# Initial kernel scaffold; baseline (speedup 1.0000x reference)
import jax
import jax.numpy as jnp
from jax.experimental import pallas as pl


def kernel(x, router, gate, up, down, shared_gate, shared_up, shared_down):
    raise NotImplementedError("write your pallas kernel here")


# speedup vs baseline: 1.9165x; 1.9165x over previous
import jax
import jax.numpy as jnp
from jax.experimental import pallas as pl
from jax.experimental.pallas import tpu as pltpu

B, T, D, F, E, TOP_K = 2, 2048, 1024, 512, 8, 2
N = B * T
BT = 512
NT = N // BT


def _silu(v):
    return v * jax.nn.sigmoid(v)


def _moe_body(x_ref, router_ref, gate_ref, up_ref, down_ref,
              sg_ref, su_ref, sd_ref, out_ref, w_scratch):
    e = pl.program_id(1)
    x = x_ref[...]

    @pl.when(e == 0)
    def _prologue():
        logits = jnp.dot(x, router_ref[...], preferred_element_type=jnp.float32)
        m = jnp.max(logits, axis=-1, keepdims=True)
        p = jnp.exp(logits - m)
        p = p / jnp.sum(p, axis=-1, keepdims=True)
        iota = jax.lax.broadcasted_iota(jnp.int32, p.shape, 1)
        m1 = jnp.max(p, axis=-1, keepdims=True)
        i1 = jnp.min(jnp.where(p == m1, iota, E), axis=-1, keepdims=True)
        p2 = jnp.where(iota == i1, -jnp.inf, p)
        m2 = jnp.max(p2, axis=-1, keepdims=True)
        i2 = jnp.min(jnp.where(p2 == m2, iota, E), axis=-1, keepdims=True)
        w_scratch[...] = jnp.where((iota == i1) | (iota == i2), p, 0.0)
        sh = jnp.dot(_silu(jnp.dot(x, sg_ref[...], preferred_element_type=jnp.float32))
                     * jnp.dot(x, su_ref[...], preferred_element_type=jnp.float32),
                     sd_ref[...], preferred_element_type=jnp.float32)
        out_ref[...] = sh

    iota = jax.lax.broadcasted_iota(jnp.int32, (BT, E), 1)
    we = jnp.sum(w_scratch[...] * (iota == e).astype(jnp.float32),
                 axis=-1, keepdims=True)
    g = jnp.dot(x, gate_ref[0], preferred_element_type=jnp.float32)
    u = jnp.dot(x, up_ref[0], preferred_element_type=jnp.float32)
    h = _silu(g) * u * we
    out_ref[...] += jnp.dot(h, down_ref[0], preferred_element_type=jnp.float32)


def kernel(x, router, gate, up, down, shared_gate, shared_up, shared_down):
    x_flat = x.reshape(N, D)
    out = pl.pallas_call(
        _moe_body,
        grid=(NT, E),
        in_specs=[
            pl.BlockSpec((BT, D), lambda i, e: (i, 0)),
            pl.BlockSpec((D, E), lambda i, e: (0, 0)),
            pl.BlockSpec((1, D, F), lambda i, e: (e, 0, 0)),
            pl.BlockSpec((1, D, F), lambda i, e: (e, 0, 0)),
            pl.BlockSpec((1, F, D), lambda i, e: (e, 0, 0)),
            pl.BlockSpec((D, F), lambda i, e: (0, 0)),
            pl.BlockSpec((D, F), lambda i, e: (0, 0)),
            pl.BlockSpec((F, D), lambda i, e: (0, 0)),
        ],
        out_specs=pl.BlockSpec((BT, D), lambda i, e: (i, 0)),
        out_shape=jax.ShapeDtypeStruct((N, D), jnp.float32),
        scratch_shapes=[pltpu.VMEM((BT, E), jnp.float32)],
    )(x_flat, router, gate, up, down, shared_gate, shared_up, shared_down)
    return out.reshape(B, T, D)


# speedup vs baseline: 1.9392x; 1.0119x over previous
import jax
import jax.numpy as jnp
from jax.experimental import pallas as pl
from jax.experimental.pallas import tpu as pltpu

B, T, D, F, E, TOP_K = 2, 2048, 1024, 512, 8, 2
N = B * T
BT = 512
NT = N // BT


def _silu(v):
    return v * jax.nn.sigmoid(v)


def _moe_body(x_ref, router_ref, gate_ref, up_ref, down_ref,
              sg_ref, su_ref, sd_ref, out_ref, w_scratch):
    e = pl.program_id(1)
    x = x_ref[...]

    @pl.when(e == 0)
    def _prologue():
        logits = jnp.dot(x, router_ref[...], preferred_element_type=jnp.float32)
        m = jnp.max(logits, axis=-1, keepdims=True)
        p = jnp.exp(logits - m)
        p = p / jnp.sum(p, axis=-1, keepdims=True)
        iota = jax.lax.broadcasted_iota(jnp.int32, p.shape, 1)
        m1 = jnp.max(p, axis=-1, keepdims=True)
        i1 = jnp.min(jnp.where(p == m1, iota, E), axis=-1, keepdims=True)
        p2 = jnp.where(iota == i1, -jnp.inf, p)
        m2 = jnp.max(p2, axis=-1, keepdims=True)
        i2 = jnp.min(jnp.where(p2 == m2, iota, E), axis=-1, keepdims=True)
        w_scratch[...] = jnp.where((iota == i1) | (iota == i2), p, 0.0)
        sh = jnp.dot(_silu(jnp.dot(x, sg_ref[...], preferred_element_type=jnp.float32))
                     * jnp.dot(x, su_ref[...], preferred_element_type=jnp.float32),
                     sd_ref[...], preferred_element_type=jnp.float32)
        out_ref[...] = sh

    iota = jax.lax.broadcasted_iota(jnp.int32, (BT, E), 1)
    we = jnp.sum(w_scratch[...] * (iota == e).astype(jnp.float32),
                 axis=-1, keepdims=True)
    xb = x.astype(jnp.bfloat16)
    g = jnp.dot(xb, gate_ref[0].astype(jnp.bfloat16),
                preferred_element_type=jnp.float32)
    u = jnp.dot(xb, up_ref[0].astype(jnp.bfloat16),
                preferred_element_type=jnp.float32)
    h = _silu(g) * u * we
    out_ref[...] += jnp.dot(h.astype(jnp.bfloat16),
                            down_ref[0].astype(jnp.bfloat16),
                            preferred_element_type=jnp.float32)


def kernel(x, router, gate, up, down, shared_gate, shared_up, shared_down):
    x_flat = x.reshape(N, D)
    out = pl.pallas_call(
        _moe_body,
        grid=(NT, E),
        in_specs=[
            pl.BlockSpec((BT, D), lambda i, e: (i, 0)),
            pl.BlockSpec((D, E), lambda i, e: (0, 0)),
            pl.BlockSpec((1, D, F), lambda i, e: (e, 0, 0)),
            pl.BlockSpec((1, D, F), lambda i, e: (e, 0, 0)),
            pl.BlockSpec((1, F, D), lambda i, e: (e, 0, 0)),
            pl.BlockSpec((D, F), lambda i, e: (0, 0)),
            pl.BlockSpec((D, F), lambda i, e: (0, 0)),
            pl.BlockSpec((F, D), lambda i, e: (0, 0)),
        ],
        out_specs=pl.BlockSpec((BT, D), lambda i, e: (i, 0)),
        out_shape=jax.ShapeDtypeStruct((N, D), jnp.float32),
        scratch_shapes=[pltpu.VMEM((BT, E), jnp.float32)],
    )(x_flat, router, gate, up, down, shared_gate, shared_up, shared_down)
    return out.reshape(B, T, D)
